# baseline (speedup 1.0000x reference)
import functools

import jax
import jax.numpy as jnp
from jax import lax
from jax.experimental import pallas as pl
from jax.experimental.pallas import tpu as pltpu
from jax.experimental.pallas import tpu_sc as plsc

_CHUNK = 32
_NBUF = 2


def _gather_kernel(n_rows, n_workers, d):
    rows_per_w = n_rows // n_workers
    n_chunks = rows_per_w // _CHUNK
    mesh = plsc.VectorSubcoreMesh(core_axis_name="c", subcore_axis_name="s")

    @functools.partial(
        pl.kernel,
        mesh=mesh,
        out_type=jax.ShapeDtypeStruct((n_rows, d), jnp.float32),
        scratch_types=[
            pltpu.VMEM((n_chunks, _CHUNK), jnp.int32),
            pltpu.VMEM((_NBUF, _CHUNK, d), jnp.float32),
            pltpu.SemaphoreType.DMA,
            pltpu.SemaphoreType.DMA,
            pltpu.SemaphoreType.DMA,
            pltpu.SemaphoreType.DMA,
            pltpu.SemaphoreType.DMA,
            pltpu.SemaphoreType.DMA,
        ],
    )
    def k(pe_hbm, idx_hbm, out_hbm, idx_v, rows_v, g0, g1, g2, s0, s1, s2):
        gsem = (g0, g1, g2)
        ssem = (s0, s1, s2)
        num_cores = lax.axis_size("c")
        wid = lax.axis_index("s") * num_cores + lax.axis_index("c")
        base = wid * rows_per_w

        def out_slice(c):
            return out_hbm.at[pl.ds(base + c * _CHUNK, _CHUNK)]

        pltpu.async_copy(idx_hbm.at[wid], idx_v, g0).wait()
        for b in range(_NBUF):
            pltpu.async_copy(pe_hbm.at[idx_v.at[b]], rows_v.at[b], gsem[b])

        for c in range(n_chunks):
            b = c % _NBUF
            pltpu.make_async_copy(
                pe_hbm.at[idx_v.at[c]], rows_v.at[b], gsem[b]
            ).wait()
            pltpu.async_copy(rows_v.at[b], out_slice(c), ssem[b])
            q = c - 1
            if q >= 0 and q + _NBUF < n_chunks:
                qb = q % _NBUF
                pltpu.make_async_copy(
                    rows_v.at[qb], out_slice(q), ssem[qb]
                ).wait()
                pltpu.async_copy(
                    pe_hbm.at[idx_v.at[q + _NBUF]], rows_v.at[qb], gsem[qb]
                )

        for q in range(n_chunks - _NBUF, n_chunks):
            qb = q % _NBUF
            pltpu.make_async_copy(rows_v.at[qb], out_slice(q), ssem[qb]).wait()

    return k


def kernel(token_positions, pe):
    b, s = token_positions.shape
    v, d = pe.shape
    n = b * s
    info = plsc.get_sparse_core_info()
    n_workers = info.num_cores * info.num_subcores
    rows_per_w = n // n_workers
    idx = token_positions.reshape(n_workers, rows_per_w // _CHUNK, _CHUNK)
    idx = idx.astype(jnp.int32)
    out = _gather_kernel(n, n_workers, d)(pe, idx)
    return out.reshape(b, s, d)


# speedup vs baseline: 1.0491x; 1.0491x over previous
import functools

import jax
import jax.numpy as jnp
from jax import lax
from jax.experimental import pallas as pl
from jax.experimental.pallas import tpu as pltpu
from jax.experimental.pallas import tpu_sc as plsc

_CHUNK = 32
_NBUF = 2


def _gather_kernel(n_rows, n_workers, d):
    rows_per_w = n_rows // n_workers
    n_chunks = rows_per_w // _CHUNK
    n_steps = n_chunks // _NBUF
    mesh = plsc.VectorSubcoreMesh(core_axis_name="c", subcore_axis_name="s")

    @functools.partial(
        pl.kernel,
        mesh=mesh,
        out_type=jax.ShapeDtypeStruct((n_rows, d), jnp.float32),
        scratch_types=[
            pltpu.VMEM((n_chunks, _CHUNK), jnp.int32),
            pltpu.VMEM((_NBUF, _CHUNK, d), jnp.float32),
            pltpu.SemaphoreType.DMA,
            pltpu.SemaphoreType.DMA,
            pltpu.SemaphoreType.DMA,
            pltpu.SemaphoreType.DMA,
        ],
    )
    def k(pe_hbm, idx_hbm, out_hbm, idx_v, rows_v, g0, g1, s0, s1):
        gsem = (g0, g1)
        ssem = (s0, s1)
        num_cores = lax.axis_size("c")
        wid = lax.axis_index("s") * num_cores + lax.axis_index("c")
        base = wid * rows_per_w
        pltpu.async_copy(idx_hbm.at[wid], idx_v, g0).wait()
        for b in range(_NBUF):
            pltpu.async_copy(pe_hbm.at[idx_v.at[b]], rows_v.at[b], gsem[b])

        def step(g, carry):
            for b in range(_NBUF):
                c = g * _NBUF + b
                pltpu.make_async_copy(
                    pe_hbm.at[idx_v.at[c]], rows_v.at[b], gsem[b]
                ).wait()
                out_slc = out_hbm.at[pl.ds(base + c * _CHUNK, _CHUNK)]
                pltpu.async_copy(rows_v.at[b], out_slc, ssem[b])

                nxt = c + _NBUF

                @pl.when(nxt < n_chunks)
                def _():
                    pltpu.make_async_copy(rows_v.at[b], out_slc, ssem[b]).wait()
                    pltpu.async_copy(
                        pe_hbm.at[idx_v.at[nxt]], rows_v.at[b], gsem[b]
                    )

            return carry

        lax.fori_loop(0, n_steps, step, 0)
        for b in range(_NBUF):
            pltpu.make_async_copy(
                rows_v.at[b], out_hbm.at[pl.ds(base, _CHUNK)], ssem[b]
            ).wait()

    return k


def kernel(token_positions, pe):
    b, s = token_positions.shape
    v, d = pe.shape
    n = b * s
    info = plsc.get_sparse_core_info()
    n_workers = info.num_cores * info.num_subcores
    rows_per_w = n // n_workers
    idx = token_positions.reshape(n_workers, rows_per_w // _CHUNK, _CHUNK)
    idx = idx.astype(jnp.int32)
    out = _gather_kernel(n, n_workers, d)(pe, idx)
    return out.reshape(b, s, d)


# speedup vs baseline: 1.0511x; 1.0019x over previous
import functools

import jax
import jax.numpy as jnp
from jax import lax
from jax.experimental import pallas as pl
from jax.experimental.pallas import tpu as pltpu
from jax.experimental.pallas import tpu_sc as plsc

_CHUNK = 16
_NBUF = 4


def _gather_kernel(n_rows, n_workers, d):
    rows_per_w = n_rows // n_workers
    n_chunks = rows_per_w // _CHUNK
    n_steps = n_chunks // _NBUF
    mesh = plsc.VectorSubcoreMesh(core_axis_name="c", subcore_axis_name="s")

    @functools.partial(
        pl.kernel,
        mesh=mesh,
        out_type=jax.ShapeDtypeStruct((n_rows, d), jnp.float32),
        scratch_types=[
            pltpu.VMEM((n_chunks, _CHUNK), jnp.int32),
            pltpu.VMEM((_NBUF, _CHUNK, d), jnp.float32),
        ] + [pltpu.SemaphoreType.DMA] * (2 * _NBUF),
    )
    def k(pe_hbm, idx_hbm, out_hbm, idx_v, rows_v, *sems):
        gsem = sems[:_NBUF]
        ssem = sems[_NBUF:]
        g0 = gsem[0]
        num_cores = lax.axis_size("c")
        wid = lax.axis_index("s") * num_cores + lax.axis_index("c")
        base = wid * rows_per_w
        pltpu.async_copy(idx_hbm.at[wid], idx_v, g0).wait()
        for b in range(_NBUF):
            pltpu.async_copy(pe_hbm.at[idx_v.at[b]], rows_v.at[b], gsem[b])

        def step(g, carry):
            for b in range(_NBUF):
                c = g * _NBUF + b
                pltpu.make_async_copy(
                    pe_hbm.at[idx_v.at[c]], rows_v.at[b], gsem[b]
                ).wait()
                out_slc = out_hbm.at[pl.ds(base + c * _CHUNK, _CHUNK)]
                pltpu.async_copy(rows_v.at[b], out_slc, ssem[b])

                nxt = c + _NBUF

                @pl.when(nxt < n_chunks)
                def _():
                    pltpu.make_async_copy(rows_v.at[b], out_slc, ssem[b]).wait()
                    pltpu.async_copy(
                        pe_hbm.at[idx_v.at[nxt]], rows_v.at[b], gsem[b]
                    )

            return carry

        lax.fori_loop(0, n_steps, step, 0)
        for b in range(_NBUF):
            pltpu.make_async_copy(
                rows_v.at[b], out_hbm.at[pl.ds(base, _CHUNK)], ssem[b]
            ).wait()

    return k


def kernel(token_positions, pe):
    b, s = token_positions.shape
    v, d = pe.shape
    n = b * s
    info = plsc.get_sparse_core_info()
    n_workers = info.num_cores * info.num_subcores
    rows_per_w = n // n_workers
    idx = token_positions.reshape(n_workers, rows_per_w // _CHUNK, _CHUNK)
    idx = idx.astype(jnp.int32)
    out = _gather_kernel(n, n_workers, d)(pe, idx)
    return out.reshape(b, s, d)


# speedup vs baseline: 1.0556x; 1.0043x over previous
import functools

import jax
import jax.numpy as jnp
from jax import lax
from jax.experimental import pallas as pl
from jax.experimental.pallas import tpu as pltpu
from jax.experimental.pallas import tpu_sc as plsc

_CHUNK = 8
_NBUF = 8


def _gather_kernel(n_rows, n_workers, d):
    rows_per_w = n_rows // n_workers
    n_chunks = rows_per_w // _CHUNK
    n_steps = n_chunks // _NBUF
    mesh = plsc.VectorSubcoreMesh(core_axis_name="c", subcore_axis_name="s")

    @functools.partial(
        pl.kernel,
        mesh=mesh,
        out_type=jax.ShapeDtypeStruct((n_rows, d), jnp.float32),
        scratch_types=[
            pltpu.VMEM((n_chunks, _CHUNK), jnp.int32),
            pltpu.VMEM((_NBUF, _CHUNK, d), jnp.float32),
        ] + [pltpu.SemaphoreType.DMA] * (2 * _NBUF),
    )
    def k(pe_hbm, idx_hbm, out_hbm, idx_v, rows_v, *sems):
        gsem = sems[:_NBUF]
        ssem = sems[_NBUF:]
        g0 = gsem[0]
        num_cores = lax.axis_size("c")
        wid = lax.axis_index("s") * num_cores + lax.axis_index("c")
        base = wid * rows_per_w
        pltpu.async_copy(idx_hbm.at[wid], idx_v, g0).wait()
        for b in range(_NBUF):
            pltpu.async_copy(pe_hbm.at[idx_v.at[b]], rows_v.at[b], gsem[b])

        def step(g, carry):
            for b in range(_NBUF):
                c = g * _NBUF + b
                pltpu.make_async_copy(
                    pe_hbm.at[idx_v.at[c]], rows_v.at[b], gsem[b]
                ).wait()
                out_slc = out_hbm.at[pl.ds(base + c * _CHUNK, _CHUNK)]
                pltpu.async_copy(rows_v.at[b], out_slc, ssem[b])

                nxt = c + _NBUF

                @pl.when(nxt < n_chunks)
                def _():
                    pltpu.make_async_copy(rows_v.at[b], out_slc, ssem[b]).wait()
                    pltpu.async_copy(
                        pe_hbm.at[idx_v.at[nxt]], rows_v.at[b], gsem[b]
                    )

            return carry

        lax.fori_loop(0, n_steps, step, 0)
        for b in range(_NBUF):
            pltpu.make_async_copy(
                rows_v.at[b], out_hbm.at[pl.ds(base, _CHUNK)], ssem[b]
            ).wait()

    return k


def kernel(token_positions, pe):
    b, s = token_positions.shape
    v, d = pe.shape
    n = b * s
    info = plsc.get_sparse_core_info()
    n_workers = info.num_cores * info.num_subcores
    rows_per_w = n // n_workers
    idx = token_positions.reshape(n_workers, rows_per_w // _CHUNK, _CHUNK)
    idx = idx.astype(jnp.int32)
    out = _gather_kernel(n, n_workers, d)(pe, idx)
    return out.reshape(b, s, d)
